# baseline (speedup 1.0000x reference)
import functools

import jax
import jax.numpy as jnp
import numpy as np
from jax import lax
from jax.experimental import pallas as pl
from jax.experimental.pallas import tpu as pltpu
from jax.experimental.pallas import tpu_sc as plsc

D_MODEL = 64
SEQ = 50
NC, NS = 2, 16
NW = NC * NS
CHUNK = 200
CB = CHUNK // SEQ
NGRP = 13
GRP = 16
LANES = 16
PAD_BLK = 4000


def _pos_encoding(max_len, d_model):
    position = jnp.arange(max_len, dtype=jnp.float32)[:, None]
    div_term = jnp.exp(
        jnp.arange(0, d_model, 2, dtype=jnp.float32) * (-np.log(10000.0) / d_model)
    )
    pe = jnp.zeros((max_len, d_model), dtype=jnp.float32)
    pe = pe.at[:, 0::2].set(jnp.sin(position * div_term))
    pe = pe.at[:, 1::2].set(jnp.cos(position * div_term))
    return pe


def _pad_table(w):
    v = w.shape[0]

    def body(in_ref, out_ref):
        out_ref[:, :D_MODEL] = in_ref[...]

    return pl.pallas_call(
        body,
        grid=(v // PAD_BLK,),
        in_specs=[pl.BlockSpec((PAD_BLK, D_MODEL), lambda i: (i, 0))],
        out_specs=pl.BlockSpec((PAD_BLK, 2 * D_MODEL), lambda i: (i, 0)),
        out_shape=jax.ShapeDtypeStruct((v, 2 * D_MODEL), jnp.float32),
    )(w)


@functools.partial(jax.jit, static_argnames=("batch", "seq"))
def _embed(idx_g, pe, table, batch, seq):
    nchunks = (batch // NW) // CB
    grp_rows = nchunks * NGRP * GRP // 128

    mesh = plsc.VectorSubcoreMesh(
        core_axis_name="c", subcore_axis_name="s", num_cores=NC, num_subcores=NS
    )

    @functools.partial(
        pl.kernel,
        out_type=jax.ShapeDtypeStruct((batch, seq, D_MODEL), jnp.float32),
        mesh=mesh,
        scratch_types=[
            pltpu.VMEM((grp_rows, 128), jnp.int32),
            pltpu.VMEM((SEQ, D_MODEL), jnp.float32),
        ]
        + [pltpu.VMEM((GRP, 2 * D_MODEL), jnp.float32) for _ in range(2 * NGRP)]
        + [pltpu.VMEM((CB, SEQ, D_MODEL), jnp.float32) for _ in range(2)]
        + [pltpu.SemaphoreType.DMA for _ in range(4)],
    )
    def body(idx_hbm, pe_hbm, table_hbm, out_hbm, idx_v, pe_v, *rest):
        gbuf = [rest[:NGRP], rest[NGRP : 2 * NGRP]]
        obuf = [rest[2 * NGRP], rest[2 * NGRP + 1]]
        gsem = [rest[2 * NGRP + 2], rest[2 * NGRP + 3]]
        osem = [rest[2 * NGRP + 4], rest[2 * NGRP + 5]]
        wid = lax.axis_index("s") * NC + lax.axis_index("c")
        pltpu.sync_copy(idx_hbm.at[wid], idx_v)
        pltpu.sync_copy(pe_hbm, pe_v)
        obatch = wid * (CB * nchunks)

        def start_gathers(c, p):
            for m in range(NGRP):
                g = c * NGRP + m
                row = lax.div(g, 8)
                col = lax.rem(g, 8) * GRP
                ivec = idx_v[row, pl.ds(col, GRP)]
                pltpu.async_copy(table_hbm.at[ivec], gbuf[p][m], gsem[p])

        def wait_gathers(p):
            for m in range(NGRP):
                pltpu.make_async_copy(
                    table_hbm.at[pl.ds(0, GRP)], gbuf[p][m], gsem[p]
                ).wait()

        def wait_store(p):
            pltpu.make_async_copy(
                obuf[p], out_hbm.at[pl.ds(0, CB)], osem[p]
            ).wait()

        start_gathers(0, 0)

        def cc_body(cc, _):
            for p in range(2):
                c = 2 * cc + p

                @pl.when(c + 1 < nchunks)
                def _():
                    start_gathers(c + 1, 1 - p)

                wait_gathers(p)

                @pl.when(c >= 2)
                def _():
                    wait_store(p)

                for m in range(NGRP):
                    nk = 4 if m < NGRP - 1 else 2

                    def k_body(k, _):
                        li = 4 * m + k
                        for bi in range(CB):
                            for j in range(D_MODEL // LANES):
                                sl = pl.ds(j * LANES, LANES)
                                obuf[p][bi, li, sl] = (
                                    gbuf[p][m][4 * k + bi, sl] + pe_v[li, sl]
                                )
                        return 0

                    lax.fori_loop(0, nk, k_body, 0)

                pltpu.async_copy(
                    obuf[p], out_hbm.at[pl.ds(obatch + CB * c, CB)], osem[p]
                )
            return 0

        lax.fori_loop(0, nchunks // 2, cc_body, 0)
        wait_store(0)
        wait_store(1)

    return body(idx_g, pe, table)


def kernel(x, W):
    batch, seq = x.shape
    pe = _pos_encoding(seq, D_MODEL)
    nchunks = (batch // NW) // CB
    xg = x.reshape(NW, nchunks, CB, seq).transpose(0, 1, 3, 2)
    xg = xg.reshape(NW, nchunks, CHUNK)
    xg = jnp.pad(xg, ((0, 0), (0, 0), (0, NGRP * GRP - CHUNK)))
    idx_g = xg.reshape(NW, nchunks * NGRP * GRP // 128, 128)
    w_pad = _pad_table(W)
    return _embed(idx_g, pe, w_pad, batch, seq)


# speedup vs baseline: 1.4426x; 1.4426x over previous
import functools

import jax
import jax.numpy as jnp
import numpy as np
from jax import lax
from jax.experimental import pallas as pl
from jax.experimental.pallas import tpu as pltpu
from jax.experimental.pallas import tpu_sc as plsc

D_MODEL = 64
SEQ = 50
NC, NS = 2, 16
NW = NC * NS
GATHER = 100
GPAD = 128
CHUNK = 200
CB = CHUNK // SEQ
LANES = 16
PACK_BLK = 4000


def _pos_encoding(max_len, d_model):
    position = jnp.arange(max_len, dtype=jnp.float32)[:, None]
    div_term = jnp.exp(
        jnp.arange(0, d_model, 2, dtype=jnp.float32) * (-np.log(10000.0) / d_model)
    )
    pe = jnp.zeros((max_len, d_model), dtype=jnp.float32)
    pe = pe.at[:, 0::2].set(jnp.sin(position * div_term))
    pe = pe.at[:, 1::2].set(jnp.cos(position * div_term))
    return pe


def _pack_table(w):
    v = w.shape[0]
    nblk = v // 2 // PACK_BLK

    def body(lo_ref, hi_ref, out_ref):
        out_ref[:, :D_MODEL] = lo_ref[...]
        out_ref[:, D_MODEL:] = hi_ref[...]

    return pl.pallas_call(
        body,
        grid=(nblk,),
        in_specs=[
            pl.BlockSpec((PACK_BLK, D_MODEL), lambda i: (i, 0)),
            pl.BlockSpec((PACK_BLK, D_MODEL), lambda i: (i + nblk, 0)),
        ],
        out_specs=pl.BlockSpec((PACK_BLK, 2 * D_MODEL), lambda i: (i, 0)),
        out_shape=jax.ShapeDtypeStruct((v // 2, 2 * D_MODEL), jnp.float32),
    )(w, w)


@functools.partial(jax.jit, static_argnames=("batch", "seq"))
def _embed(idxpar, pe, wp, batch, seq):
    nchunks = (batch // NW) // CB
    ngather = 2 * nchunks

    mesh = plsc.VectorSubcoreMesh(
        core_axis_name="c", subcore_axis_name="s", num_cores=NC, num_subcores=NS
    )

    @functools.partial(
        pl.kernel,
        out_type=jax.ShapeDtypeStruct((batch, seq, D_MODEL), jnp.float32),
        mesh=mesh,
        scratch_types=[
            pltpu.VMEM((SEQ, D_MODEL), jnp.float32),
        ]
        + [pltpu.VMEM((2, GPAD), jnp.int32) for _ in range(4)]
        + [pltpu.VMEM((GATHER, 2 * D_MODEL), jnp.float32) for _ in range(4)]
        + [pltpu.VMEM((CB, SEQ, D_MODEL), jnp.float32) for _ in range(2)]
        + [pltpu.SemaphoreType.DMA for _ in range(6)],
    )
    def body(idx_hbm, pe_hbm, table_hbm, out_hbm, pe_v, *rest):
        ring = [rest[0:2], rest[2:4]]
        gbuf = [rest[4:6], rest[6:8]]
        obuf = [rest[8], rest[9]]
        isem = [rest[10], rest[11]]
        gsem = [rest[12], rest[13]]
        osem = [rest[14], rest[15]]
        wid = lax.axis_index("s") * NC + lax.axis_index("c")
        pltpu.sync_copy(pe_hbm, pe_v)
        obatch = wid * (CB * nchunks)

        def stage_idx(c, p):
            for h in range(2):
                pltpu.async_copy(idx_hbm.at[wid, 2 * c + h], ring[p][h], isem[p])

        def wait_idx(p):
            for h in range(2):
                pltpu.make_async_copy(
                    idx_hbm.at[wid, h], ring[p][h], isem[p]
                ).wait()

        def start_gathers(p):
            for h in range(2):
                pltpu.async_copy(
                    table_hbm.at[ring[p][h].at[0, pl.ds(0, GATHER)]],
                    gbuf[p][h],
                    gsem[p],
                )

        def wait_gathers(p):
            for h in range(2):
                pltpu.make_async_copy(
                    table_hbm.at[ring[p][h].at[0, pl.ds(0, GATHER)]],
                    gbuf[p][h],
                    gsem[p],
                ).wait()

        def wait_store(p):
            pltpu.make_async_copy(
                obuf[p], out_hbm.at[pl.ds(0, CB)], osem[p]
            ).wait()

        stage_idx(0, 0)
        stage_idx(1, 1)
        wait_idx(0)
        start_gathers(0)

        def cc_body(cc, _):
            for p in range(2):
                c = 2 * cc + p

                @pl.when(c + 1 < nchunks)
                def _():
                    wait_idx(1 - p)
                    start_gathers(1 - p)

                wait_gathers(p)

                @pl.when(c >= 2)
                def _():
                    wait_store(p)

                for bi in range(CB):
                    h = bi // 2
                    base_i = (bi % 2) * SEQ

                    def li_body(li, _):
                        i = base_i + li
                        lane = lax.rem(i, LANES)
                        col = i - lane
                        parv = ring[p][h][1, pl.ds(col, LANES)]
                        pb = parv[jnp.full((LANES,), lane, jnp.int32)].astype(
                            jnp.float32
                        )
                        for j in range(D_MODEL // LANES):
                            sl = pl.ds(j * LANES, LANES)
                            slr = pl.ds(D_MODEL + j * LANES, LANES)
                            a = gbuf[p][h][i, sl]
                            b = gbuf[p][h][i, slr]
                            obuf[p][bi, li, sl] = (
                                a + pb * (b - a) + pe_v[li, sl]
                            )
                        return 0

                    lax.fori_loop(0, SEQ, li_body, 0)

                pltpu.async_copy(
                    obuf[p], out_hbm.at[pl.ds(obatch + CB * c, CB)], osem[p]
                )

                @pl.when(c + 2 < nchunks)
                def _():
                    stage_idx(c + 2, p)
            return 0

        lax.fori_loop(0, nchunks // 2, cc_body, 0)
        wait_store(0)
        wait_store(1)

    return body(idxpar, pe, wp)


def kernel(x, W):
    batch, seq = x.shape
    pe = _pos_encoding(seq, D_MODEL)
    nchunks = (batch // NW) // CB
    xw = x.reshape(NW, 2 * nchunks, GATHER)
    half = W.shape[0] // 2
    rows = jnp.pad(xw % half, ((0, 0), (0, 0), (0, GPAD - GATHER)))
    pars = jnp.pad(xw // half, ((0, 0), (0, 0), (0, GPAD - GATHER)))
    idxpar = jnp.stack([rows, pars], axis=2)
    wp = _pack_table(W)
    return _embed(idxpar, pe, wp, batch, seq)
